# Initial kernel scaffold; baseline (speedup 1.0000x reference)
import functools

import jax
import jax.numpy as jnp
from jax import lax
from jax.experimental import pallas as pl
from jax.experimental.pallas import tpu as pltpu
from jax.experimental.pallas import tpu_sc as plsc

EPS = 1e-8
BIG = jnp.float32(1e30)
BIG_I = jnp.int32(2**30)
W = 128
KB = 2048
QT = 256
TOPK = 20


def _pc1_body(nkb, K, q_hbm, k_hbm, sims_ref, bmin_ref, qn_s, kn_s):
    jk = pl.program_id(0)
    iq = pl.program_id(1)

    @pl.when(jnp.logical_and(jk == 0, iq == 0))
    def _():
        q = q_hbm[...]
        qn = jnp.sqrt(jnp.sum(q * q, axis=1, keepdims=True))
        qn_s[...] = q / jnp.maximum(qn, EPS)

    @pl.when(iq == 0)
    def _():
        kb = k_hbm[...]
        kn = jnp.sqrt(jnp.sum(kb * kb, axis=1, keepdims=True))
        kn_s[...] = kb / jnp.maximum(kn, EPS)

    QTC = q_hbm.shape[0] // pl.num_programs(1)
    qn_t = qn_s[pl.ds(iq * QTC, QTC), :]
    sims = jax.lax.dot_general(
        qn_t, kn_s[...],
        dimension_numbers=(((1,), (1,)), ((), ())),
        preferred_element_type=jnp.float32,
    )
    col = jk * KB + jax.lax.broadcasted_iota(jnp.int32, sims.shape, 1)
    sims = jnp.where(col < K, sims, BIG)
    sims_ref[...] = sims
    bmin_ref[...] = jnp.min(
        sims.reshape(QTC, KB // W, W), axis=2
    ).T


def _make_pc2(nblk_pad, nblk):
    def body(bmin_ref, sel_ref, gidx_ref, x_s):
        i = pl.program_id(0)
        x_s[...] = bmin_ref[...]
        sel_ref[...] = jnp.zeros(sel_ref.shape, jnp.int32)
        gidx_ref[...] = jnp.zeros(gidx_ref.shape, jnp.int32)
        nq_t = x_s.shape[0]
        iota1 = jax.lax.broadcasted_iota(jnp.int32, (nq_t, nblk_pad), 1)
        rowq = (i * nq_t
                + jax.lax.broadcasted_iota(jnp.int32, (nq_t, 1), 0)) * nblk
        for j in range(TOPK):
            x = x_s[...]
            m = jnp.min(x, axis=1, keepdims=True)
            amin = jnp.min(
                jnp.where(x == m, iota1, BIG_I), axis=1, keepdims=True)
            sel_ref[:, pl.ds(j, 1)] = amin
            gidx_ref[:, pl.ds(j, 1)] = rowq + amin
            x_s[...] = jnp.where(iota1 == amin, BIG, x)
    return body


def _pc4_body(gat_ref, sel_ref, vals_ref, idx_ref, x_s):
    CW = TOPK * W
    nq_t = gat_ref.shape[0]
    x_s[...] = gat_ref[...]
    vals_ref[...] = jnp.zeros(vals_ref.shape, jnp.float32)
    idx_ref[...] = jnp.zeros(idx_ref.shape, jnp.int32)
    iota1 = jax.lax.broadcasted_iota(jnp.int32, (nq_t, CW), 1)
    for j in range(TOPK):
        x = x_s[...]
        m = jnp.min(x, axis=1, keepdims=True)
        amin = jnp.min(
            jnp.where(x == m, iota1, BIG_I), axis=1, keepdims=True)
        bj = amin // W
        within = amin - bj * W
        selblk = jnp.zeros((nq_t, 1), jnp.int32)
        for jj in range(TOPK):
            selblk = jnp.where(bj == jj, sel_ref[:, pl.ds(jj, 1)], selblk)
        vals_ref[:, pl.ds(j, 1)] = m
        idx_ref[:, pl.ds(j, 1)] = selblk * W + within
        x_s[...] = jnp.where(iota1 == amin, BIG, x)


def _sc_gather(sims_flat, gidx, n_rows):
    info = plsc.get_sparse_core_info()
    NC, NS = info.num_cores, info.num_subcores
    NW = NC * NS
    b_per_w = n_rows // NW
    CH = 128
    n_ch = b_per_w // CH
    mesh = plsc.VectorSubcoreMesh(core_axis_name="c", subcore_axis_name="s")

    @functools.partial(
        pl.kernel, mesh=mesh,
        out_type=jax.ShapeDtypeStruct((n_rows, W), jnp.float32),
        scratch_types=[
            pltpu.VMEM((b_per_w,), jnp.int32),
            pltpu.VMEM((CH, W), jnp.float32),
            pltpu.SemaphoreType.DMA,
        ],
    )
    def k(table_hbm, idx_hbm, out_hbm, idx_v, rows_v, sem):
        wid = lax.axis_index("s") * NC + lax.axis_index("c")
        base = wid * b_per_w
        pltpu.sync_copy(idx_hbm.at[pl.ds(base, b_per_w)], idx_v)

        def chunk(c, carry):
            pltpu.async_copy(
                table_hbm.at[idx_v.at[pl.ds(c * CH, CH)]], rows_v, sem
            ).wait()
            pltpu.sync_copy(rows_v, out_hbm.at[pl.ds(base + c * CH, CH)])
            return carry

        lax.fori_loop(0, n_ch, chunk, 0)

    return k(sims_flat, gidx)


def kernel(queries, keys, k):
    Q, D = queries.shape
    K = keys.shape[0]
    nkb = -(-K // KB)
    Kpad = nkb * KB
    nblk = nkb * (KB // W)
    nblk_pad = -(-nblk // 128) * 128
    nq = Q // QT

    keys_p = jnp.pad(keys, ((0, Kpad - K), (0, 0)))

    sims, bminT = pl.pallas_call(
        functools.partial(_pc1_body, nkb, K),
        grid=(nkb, nq),
        in_specs=[
            pl.BlockSpec((Q, D), lambda jk, iq: (0, 0)),
            pl.BlockSpec((KB, D), lambda jk, iq: (jk, 0)),
        ],
        out_specs=[
            pl.BlockSpec((QT, KB), lambda jk, iq: (iq, jk)),
            pl.BlockSpec((KB // W, QT), lambda jk, iq: (jk, iq)),
        ],
        out_shape=[
            jax.ShapeDtypeStruct((Q, Kpad), jnp.float32),
            jax.ShapeDtypeStruct((nblk, Q), jnp.float32),
        ],
        scratch_shapes=[
            pltpu.VMEM((Q, D), jnp.float32),
            pltpu.VMEM((KB, D), jnp.float32),
        ],
        compiler_params=pltpu.CompilerParams(
            dimension_semantics=("arbitrary", "arbitrary"),
        ),
    )(queries, keys_p)

    bmin = jnp.pad(bminT.T, ((0, 0), (0, nblk_pad - nblk)),
                   constant_values=BIG)

    sel_p, gidx_p = pl.pallas_call(
        _make_pc2(nblk_pad, nblk),
        grid=(nq,),
        in_specs=[pl.BlockSpec((QT, nblk_pad), lambda i: (i, 0))],
        out_specs=[
            pl.BlockSpec((QT, 128), lambda i: (i, 0)),
            pl.BlockSpec((QT, 128), lambda i: (i, 0)),
        ],
        out_shape=[
            jax.ShapeDtypeStruct((Q, 128), jnp.int32),
            jax.ShapeDtypeStruct((Q, 128), jnp.int32),
        ],
        scratch_shapes=[pltpu.VMEM((QT, nblk_pad), jnp.float32)],
    )(bmin)

    gidx = gidx_p[:, :TOPK].reshape(Q * TOPK)
    sims_flat = sims.reshape(Q * nblk, W)
    gat = _sc_gather(sims_flat, gidx, Q * TOPK)
    gat2 = gat.reshape(Q, TOPK * W)

    vals_p, idx_p = pl.pallas_call(
        _pc4_body,
        grid=(nq,),
        in_specs=[
            pl.BlockSpec((QT, TOPK * W), lambda i: (i, 0)),
            pl.BlockSpec((QT, 128), lambda i: (i, 0)),
        ],
        out_specs=[
            pl.BlockSpec((QT, 128), lambda i: (i, 0)),
            pl.BlockSpec((QT, 128), lambda i: (i, 0)),
        ],
        out_shape=[
            jax.ShapeDtypeStruct((Q, 128), jnp.float32),
            jax.ShapeDtypeStruct((Q, 128), jnp.int32),
        ],
        scratch_shapes=[pltpu.VMEM((QT, TOPK * W), jnp.float32)],
    )(gat2, sel_p)

    return vals_p[:, :TOPK], idx_p[:, :TOPK]


# speedup vs baseline: 6.4946x; 6.4946x over previous
import functools

import jax
import jax.numpy as jnp
from jax import lax
from jax.experimental import pallas as pl
from jax.experimental.pallas import tpu as pltpu
from jax.experimental.pallas import tpu_sc as plsc

EPS = 1e-8
BIG = 1e30
BIG_I = 2**30
W = 128
KB = 2048
QT = 256
TOPK = 20


def _pc1_body(nkb, K, q_hbm, k_hbm, sims_ref, bmin_ref, qn_s, kn_s):
    jk = pl.program_id(0)
    iq = pl.program_id(1)

    @pl.when(jnp.logical_and(jk == 0, iq == 0))
    def _():
        q = q_hbm[...]
        qn = jnp.sqrt(jnp.sum(q * q, axis=1, keepdims=True))
        qn_s[...] = q / jnp.maximum(qn, EPS)

    @pl.when(iq == 0)
    def _():
        kb = k_hbm[...]
        kn = jnp.sqrt(jnp.sum(kb * kb, axis=1, keepdims=True))
        kn_s[...] = kb / jnp.maximum(kn, EPS)

    QTC = q_hbm.shape[0] // pl.num_programs(1)
    qn_t = qn_s[pl.ds(iq * QTC, QTC), :]
    sims = jax.lax.dot_general(
        qn_t, kn_s[...],
        dimension_numbers=(((1,), (1,)), ((), ())),
        preferred_element_type=jnp.float32,
    )
    col = jk * KB + jax.lax.broadcasted_iota(jnp.int32, sims.shape, 1)
    sims = jnp.where(col < K, sims, BIG)
    sims_ref[...] = sims
    bmin_ref[...] = jnp.min(
        sims.reshape(QTC, KB // W, W), axis=2
    ).T


def _make_pc2(nblk_pad, nblk):
    def body(bmin_ref, sel_ref, gidx_ref, x_s):
        i = pl.program_id(0)
        x_s[...] = bmin_ref[...]
        sel_ref[...] = jnp.zeros(sel_ref.shape, jnp.int32)
        gidx_ref[...] = jnp.zeros(gidx_ref.shape, jnp.int32)
        nq_t = x_s.shape[0]
        iota1 = jax.lax.broadcasted_iota(jnp.int32, (nq_t, nblk_pad), 1)
        rowq = (i * nq_t
                + jax.lax.broadcasted_iota(jnp.int32, (nq_t, 1), 0)) * nblk
        for j in range(TOPK):
            x = x_s[...]
            m = jnp.min(x, axis=1, keepdims=True)
            amin = jnp.min(
                jnp.where(x == m, iota1, BIG_I), axis=1, keepdims=True)
            sel_ref[:, pl.ds(j, 1)] = amin
            gidx_ref[:, pl.ds(j, 1)] = rowq + amin
            x_s[...] = jnp.where(iota1 == amin, BIG, x)
    return body


def _pc4_body(gat_ref, sel_ref, vals_ref, idx_ref, x_s):
    CW = TOPK * W
    nq_t = gat_ref.shape[0]
    x_s[...] = gat_ref[...]
    vals_ref[...] = jnp.zeros(vals_ref.shape, jnp.float32)
    idx_ref[...] = jnp.zeros(idx_ref.shape, jnp.int32)
    iota1 = jax.lax.broadcasted_iota(jnp.int32, (nq_t, CW), 1)
    for j in range(TOPK):
        x = x_s[...]
        m = jnp.min(x, axis=1, keepdims=True)
        amin = jnp.min(
            jnp.where(x == m, iota1, BIG_I), axis=1, keepdims=True)
        bj = amin // W
        within = amin - bj * W
        selblk = jnp.zeros((nq_t, 1), jnp.int32)
        for jj in range(TOPK):
            selblk = jnp.where(bj == jj, sel_ref[:, pl.ds(jj, 1)], selblk)
        vals_ref[:, pl.ds(j, 1)] = m
        idx_ref[:, pl.ds(j, 1)] = selblk * W + within
        x_s[...] = jnp.where(iota1 == amin, BIG, x)


def _sc_gather(sims_flat, gidx, n_rows):
    info = plsc.get_sparse_core_info()
    NC, NS = info.num_cores, info.num_subcores
    NW = NC * NS
    b_per_w = n_rows // NW
    CH = 128
    n_ch = b_per_w // CH
    mesh = plsc.VectorSubcoreMesh(core_axis_name="c", subcore_axis_name="s")

    @functools.partial(
        pl.kernel, mesh=mesh,
        out_type=jax.ShapeDtypeStruct((n_rows, W), jnp.float32),
        scratch_types=[
            pltpu.VMEM((b_per_w,), jnp.int32),
            pltpu.VMEM((CH, W), jnp.float32),
            pltpu.SemaphoreType.DMA,
        ],
    )
    def k(table_hbm, idx_hbm, out_hbm, idx_v, rows_v, sem):
        wid = lax.axis_index("s") * NC + lax.axis_index("c")
        base = wid * b_per_w
        pltpu.sync_copy(idx_hbm.at[pl.ds(base, b_per_w)], idx_v)

        def chunk(c, carry):
            pltpu.async_copy(
                table_hbm.at[idx_v.at[pl.ds(c * CH, CH)]], rows_v, sem
            ).wait()
            pltpu.sync_copy(rows_v, out_hbm.at[pl.ds(base + c * CH, CH)])
            return carry

        lax.fori_loop(0, n_ch, chunk, 0)

    return k(sims_flat, gidx)


def kernel(queries, keys, k):
    Q, D = queries.shape
    K = keys.shape[0]
    nkb = -(-K // KB)
    Kpad = nkb * KB
    nblk = nkb * (KB // W)
    nblk_pad = -(-nblk // 128) * 128
    nq = Q // QT

    keys_p = jnp.pad(keys, ((0, Kpad - K), (0, 0)))

    sims, bminT = pl.pallas_call(
        functools.partial(_pc1_body, nkb, K),
        grid=(nkb, nq),
        in_specs=[
            pl.BlockSpec((Q, D), lambda jk, iq: (0, 0)),
            pl.BlockSpec((KB, D), lambda jk, iq: (jk, 0)),
        ],
        out_specs=[
            pl.BlockSpec((QT, KB), lambda jk, iq: (iq, jk)),
            pl.BlockSpec((KB // W, QT), lambda jk, iq: (jk, iq)),
        ],
        out_shape=[
            jax.ShapeDtypeStruct((Q, Kpad), jnp.float32),
            jax.ShapeDtypeStruct((nblk, Q), jnp.float32),
        ],
        scratch_shapes=[
            pltpu.VMEM((Q, D), jnp.float32),
            pltpu.VMEM((KB, D), jnp.float32),
        ],
        compiler_params=pltpu.CompilerParams(
            dimension_semantics=("arbitrary", "arbitrary"),
        ),
    )(queries, keys_p)

    bmin = jnp.pad(bminT.T, ((0, 0), (0, nblk_pad - nblk)),
                   constant_values=BIG)

    sel_p, gidx_p = pl.pallas_call(
        _make_pc2(nblk_pad, nblk),
        grid=(nq,),
        in_specs=[pl.BlockSpec((QT, nblk_pad), lambda i: (i, 0))],
        out_specs=[
            pl.BlockSpec((QT, 128), lambda i: (i, 0)),
            pl.BlockSpec((QT, 128), lambda i: (i, 0)),
        ],
        out_shape=[
            jax.ShapeDtypeStruct((Q, 128), jnp.int32),
            jax.ShapeDtypeStruct((Q, 128), jnp.int32),
        ],
        scratch_shapes=[pltpu.VMEM((QT, nblk_pad), jnp.float32)],
    )(bmin)

    gidx = gidx_p[:, :TOPK].reshape(Q * TOPK)
    sims_flat = sims.reshape(Q * nblk, W)
    gat = _sc_gather(sims_flat, gidx, Q * TOPK)
    gat2 = gat.reshape(Q, TOPK * W)

    vals_p, idx_p = pl.pallas_call(
        _pc4_body,
        grid=(nq,),
        in_specs=[
            pl.BlockSpec((QT, TOPK * W), lambda i: (i, 0)),
            pl.BlockSpec((QT, 128), lambda i: (i, 0)),
        ],
        out_specs=[
            pl.BlockSpec((QT, 128), lambda i: (i, 0)),
            pl.BlockSpec((QT, 128), lambda i: (i, 0)),
        ],
        out_shape=[
            jax.ShapeDtypeStruct((Q, 128), jnp.float32),
            jax.ShapeDtypeStruct((Q, 128), jnp.int32),
        ],
        scratch_shapes=[pltpu.VMEM((QT, TOPK * W), jnp.float32)],
    )(gat2, sel_p)

    return vals_p[:, :TOPK], idx_p[:, :TOPK]
